# baseline (speedup 1.0000x reference)
import functools

import jax
import jax.numpy as jnp
from jax import lax
from jax.experimental import pallas as pl
from jax.experimental.pallas import tpu as pltpu
from jax.experimental.pallas import tpu_sc as plsc

_NC = 2
_NS = 16
_NW = _NC * _NS

_LANES = 16


def _node_tables_kernel(x_ref, w1_ref, w2_ref, a_ref, b_ref):
    xv = x_ref[...]
    a_ref[...] = jnp.dot(xv, w1_ref[...], preferred_element_type=jnp.float32)
    b_ref[...] = jnp.dot(xv, w2_ref[...], preferred_element_type=jnp.float32)


def _edge_final_kernel(g_ref, ea_ref, w3_ref, bias_ref, out_ref):
    hb = g_ref.shape[0]
    gi = lax.bitcast_convert_type(g_ref[...], jnp.uint32)
    lo = lax.bitcast_convert_type(gi << 16, jnp.float32)
    hi = lax.bitcast_convert_type(gi & jnp.uint32(0xFFFF0000), jnp.float32)
    even = jnp.concatenate([lo[:, :64], hi[:, :64]], axis=1)
    odd = jnp.concatenate([lo[:, 64:], hi[:, 64:]], axis=1)
    g = jnp.stack([even, odd], axis=1).reshape(2 * hb, 128)
    out_ref[...] = (
        g
        + jnp.dot(ea_ref[...], w3_ref[...], preferred_element_type=jnp.float32)
        + bias_ref[...]
    )


def _make_sc_gather_add(E, D, CH):
    per_w = E // _NW
    nchunk = per_w // CH
    mesh = plsc.VectorSubcoreMesh(core_axis_name="c", subcore_axis_name="s")

    @functools.partial(
        pl.kernel,
        out_type=jax.ShapeDtypeStruct((E // 2, D), jnp.float32),
        mesh=mesh,
        scratch_types=[
            pltpu.VMEM((CH,), jnp.int32),
            pltpu.VMEM((CH,), jnp.int32),
            pltpu.VMEM((CH,), jnp.int32),
            pltpu.VMEM((CH,), jnp.int32),
            pltpu.VMEM((CH, D), jnp.float32),
            pltpu.VMEM((CH, D), jnp.float32),
            pltpu.VMEM((CH, D), jnp.float32),
            pltpu.VMEM((CH, D), jnp.float32),
            pltpu.VMEM((CH // 2, D), jnp.float32),
            pltpu.VMEM((CH // 2, D), jnp.float32),
        ] + [pltpu.SemaphoreType.DMA] * 10,
    )
    def sc_edge_kernel(a_hbm, b_hbm, row_hbm, col_hbm, out_hbm,
                       row0, col0, row1, col1, bufa0, bufb0, bufa1, bufb1,
                       bufg0, bufg1,
                       semr0, semc0, semr1, semc1,
                       sema0, semb0, sema1, semb1, semst0, semst1):
        rowv = (row0, row1)
        colv = (col0, col1)
        bufa = (bufa0, bufa1)
        bufb = (bufb0, bufb1)
        bufg = (bufg0, bufg1)
        semr = (semr0, semr1)
        semc = (semc0, semc1)
        sema = (sema0, sema1)
        semb = (semb0, semb1)
        semst = (semst0, semst1)

        wid = lax.axis_index("s") * _NC + lax.axis_index("c")
        base_w = wid * per_w
        base_h = wid * (per_w // 2)
        chh = CH // 2
        hi_mask = jnp.uint32(0xFFFF0000)

        def issue_idx(i, p):
            base = base_w + i * CH
            pltpu.async_copy(row_hbm.at[pl.ds(base, CH)], rowv[p], semr[p])
            pltpu.async_copy(col_hbm.at[pl.ds(base, CH)], colv[p], semc[p])

        def wait_idx(p):
            pltpu.make_async_copy(
                row_hbm.at[pl.ds(0, CH)], rowv[p], semr[p]).wait()
            pltpu.make_async_copy(
                col_hbm.at[pl.ds(0, CH)], colv[p], semc[p]).wait()

        def issue_gather(p):
            pltpu.async_copy(a_hbm.at[rowv[p]], bufa[p], sema[p])
            pltpu.async_copy(b_hbm.at[colv[p]], bufb[p], semb[p])

        def wait_gather(p):
            pltpu.make_async_copy(a_hbm.at[rowv[p]], bufa[p], sema[p]).wait()
            pltpu.make_async_copy(b_hbm.at[colv[p]], bufb[p], semb[p]).wait()

        def issue_store(i, p):
            pltpu.async_copy(
                bufg[p], out_hbm.at[pl.ds(base_h + i * chh, chh)], semst[p])

        def wait_store(p):
            pltpu.make_async_copy(
                bufg[p], out_hbm.at[pl.ds(0, chh)], semst[p]).wait()

        def compute(p):
            ba, bb, bg = bufa[p], bufb[p], bufg[p]

            def pair_body(m, c2):
                for half in range(2):
                    e = 2 * m + half
                    for j in range(4):
                        sll = pl.ds(j * _LANES, _LANES)
                        slh = pl.ds(64 + j * _LANES, _LANES)
                        vlo = ba[e, sll] + bb[e, sll]
                        vhi = ba[e, slh] + bb[e, slh]
                        w = ((lax.bitcast_convert_type(vlo, jnp.uint32) >> 16)
                             | (lax.bitcast_convert_type(vhi, jnp.uint32)
                                & hi_mask))
                        bg[m, pl.ds(half * 64 + j * _LANES, _LANES)] = (
                            lax.bitcast_convert_type(w, jnp.float32))
                return c2

            lax.fori_loop(0, chh, pair_body, 0, unroll=False)

        def step(i, p, store_wait, gather_next, idx_next):
            if gather_next:
                if store_wait:
                    wait_store(1 - p)
                wait_idx(1 - p)
                issue_gather(1 - p)
            wait_gather(p)
            if idx_next:
                issue_idx(i + 2, p)
            compute(p)
            issue_store(i, p)

        assert nchunk >= 6 and nchunk % 2 == 1
        issue_idx(0, 0)
        issue_idx(1, 1)
        wait_idx(0)
        issue_gather(0)

        step(0, 0, False, True, True)
        step(1, 1, True, True, True)

        def body(g, carry):
            i = 2 * g + 2
            step(i, 0, True, True, True)
            step(i + 1, 1, True, True, True)
            return carry

        npairs = (nchunk - 5) // 2
        lax.fori_loop(0, npairs, body, 0, unroll=False)

        step(nchunk - 3, 0, True, True, True)
        step(nchunk - 2, 1, True, True, False)
        step(nchunk - 1, 0, True, False, False)

        wait_store(1)
        wait_store(0)

    return sc_edge_kernel


def kernel(x, edge_index, edge_attr, edge_type, W, b):
    del edge_type
    N, D = x.shape
    E, DE = edge_attr.shape
    DOUT = W.shape[1]

    W1 = lax.slice(W, (0, 0), (D, DOUT))
    W2 = lax.slice(W, (D, 0), (2 * D, DOUT))
    W3 = lax.slice(W, (2 * D, 0), (2 * D + DE, DOUT))
    row = edge_index[0]
    col = edge_index[1]

    A, B = pl.pallas_call(
        _node_tables_kernel,
        out_shape=[
            jax.ShapeDtypeStruct((N, DOUT), jnp.float32),
            jax.ShapeDtypeStruct((N, DOUT), jnp.float32),
        ],
    )(x, W1, W2)

    sc = _make_sc_gather_add(E, DOUT, CH=80)
    G = sc(A, B, row, col)

    BE = 3200
    grid = (E // BE,)
    out = pl.pallas_call(
        _edge_final_kernel,
        grid=grid,
        in_specs=[
            pl.BlockSpec((BE // 2, DOUT), lambda i: (i, 0)),
            pl.BlockSpec((BE, DE), lambda i: (i, 0)),
            pl.BlockSpec((DE, DOUT), lambda i: (0, 0)),
            pl.BlockSpec((1, DOUT), lambda i: (0, 0)),
        ],
        out_specs=pl.BlockSpec((BE, DOUT), lambda i: (i, 0)),
        out_shape=jax.ShapeDtypeStruct((E, DOUT), jnp.float32),
    )(G, edge_attr, W3, b.reshape(1, DOUT))
    return out


# speedup vs baseline: 1.4579x; 1.4579x over previous
import functools

import jax
import jax.numpy as jnp
from jax import lax
from jax.experimental import pallas as pl
from jax.experimental.pallas import tpu as pltpu
from jax.experimental.pallas import tpu_sc as plsc

_NC = 2
_NS = 16
_NW = _NC * _NS

_LANES = 16


def _node_tables_kernel(x_ref, w1_ref, w2_ref, a_ref, b_ref):
    xv = x_ref[...]
    a_ref[...] = jnp.dot(xv, w1_ref[...], preferred_element_type=jnp.float32)
    b_ref[...] = jnp.dot(xv, w2_ref[...], preferred_element_type=jnp.float32)


def _edge_final_kernel(g_ref, ea_ref, w3_ref, bias_ref, out_ref):
    out_ref[...] = (
        g_ref[...]
        + jnp.dot(ea_ref[...], w3_ref[...], preferred_element_type=jnp.float32)
        + bias_ref[...]
    )


def _edge_final_kernel2(g_ref, ea_ref, w3_ref, bias_ref, prev_ref, out_ref):
    del prev_ref
    out_ref[...] = (
        g_ref[...]
        + jnp.dot(ea_ref[...], w3_ref[...], preferred_element_type=jnp.float32)
        + bias_ref[...]
    )


def _make_sc_gather_add(EH, D, CH):
    per_w = EH // _NW
    nchunk = per_w // CH
    jperrow = D // _LANES
    mesh = plsc.VectorSubcoreMesh(core_axis_name="c", subcore_axis_name="s")

    @functools.partial(
        pl.kernel,
        out_type=jax.ShapeDtypeStruct((EH, D), jnp.float32),
        mesh=mesh,
        scratch_types=[
            pltpu.VMEM((CH,), jnp.int32),
            pltpu.VMEM((CH,), jnp.int32),
            pltpu.VMEM((CH,), jnp.int32),
            pltpu.VMEM((CH,), jnp.int32),
            pltpu.VMEM((CH, D), jnp.float32),
            pltpu.VMEM((CH, D), jnp.float32),
            pltpu.VMEM((CH, D), jnp.float32),
            pltpu.VMEM((CH, D), jnp.float32),
        ] + [pltpu.SemaphoreType.DMA] * 10,
    )
    def sc_edge_kernel(a_hbm, b_hbm, row_hbm, col_hbm, out_hbm,
                       row0, col0, row1, col1, bufa0, bufb0, bufa1, bufb1,
                       semr0, semc0, semr1, semc1,
                       sema0, semb0, sema1, semb1, semst0, semst1):
        rowv = (row0, row1)
        colv = (col0, col1)
        bufa = (bufa0, bufa1)
        bufb = (bufb0, bufb1)
        semr = (semr0, semr1)
        semc = (semc0, semc1)
        sema = (sema0, sema1)
        semb = (semb0, semb1)
        semst = (semst0, semst1)

        wid = lax.axis_index("s") * _NC + lax.axis_index("c")
        base_w = wid * per_w

        def issue_idx(i):
            p = i % 2
            base = base_w + i * CH
            return (
                pltpu.async_copy(row_hbm.at[pl.ds(base, CH)], rowv[p], semr[p]),
                pltpu.async_copy(col_hbm.at[pl.ds(base, CH)], colv[p], semc[p]),
            )

        def issue_gather(i):
            p = i % 2
            return (
                pltpu.async_copy(a_hbm.at[rowv[p]], bufa[p], sema[p]),
                pltpu.async_copy(b_hbm.at[colv[p]], bufb[p], semb[p]),
            )

        idxd = [None] * (nchunk + 1)
        gd = [None] * (nchunk + 1)
        std = [None] * (nchunk + 1)

        idxd[0] = issue_idx(0)
        if nchunk > 1:
            idxd[1] = issue_idx(1)
        idxd[0][0].wait()
        idxd[0][1].wait()
        gd[0] = issue_gather(0)

        for i in range(nchunk):
            p = i % 2
            if i + 1 < nchunk:
                if i >= 1:
                    std[i - 1].wait()
                idxd[i + 1][0].wait()
                idxd[i + 1][1].wait()
                gd[i + 1] = issue_gather(i + 1)
            gd[i][0].wait()
            gd[i][1].wait()
            if i + 2 < nchunk:
                idxd[i + 2] = issue_idx(i + 2)

            ba, bb = bufa[p], bufb[p]

            def add_body(e, c2):
                for j in range(jperrow):
                    sl = pl.ds(j * _LANES, _LANES)
                    ba[e, sl] = ba[e, sl] + bb[e, sl]
                return c2

            lax.fori_loop(0, CH, add_body, 0, unroll=False)
            std[i] = pltpu.async_copy(
                ba, out_hbm.at[pl.ds(base_w + i * CH, CH)], semst[p])

        if nchunk >= 2:
            std[nchunk - 2].wait()
        std[nchunk - 1].wait()

    return sc_edge_kernel


def kernel(x, edge_index, edge_attr, edge_type, W, b):
    del edge_type
    N, D = x.shape
    E, DE = edge_attr.shape
    DOUT = W.shape[1]
    EH = E // 2

    W1 = lax.slice(W, (0, 0), (D, DOUT))
    W2 = lax.slice(W, (D, 0), (2 * D, DOUT))
    W3 = lax.slice(W, (2 * D, 0), (2 * D + DE, DOUT))
    b2 = b.reshape(1, DOUT)
    row = edge_index[0]
    col = edge_index[1]
    row0 = lax.slice(row, (0,), (EH,))
    col0 = lax.slice(col, (0,), (EH,))
    row1 = lax.slice(row, (EH,), (E,))
    col1 = lax.slice(col, (EH,), (E,))

    A, B = pl.pallas_call(
        _node_tables_kernel,
        out_shape=[
            jax.ShapeDtypeStruct((N, DOUT), jnp.float32),
            jax.ShapeDtypeStruct((N, DOUT), jnp.float32),
        ],
    )(x, W1, W2)

    sc = _make_sc_gather_add(EH, DOUT, CH=200)
    G0 = sc(A, B, row0, col0)
    G1 = sc(A, B, row1, col1)

    BE = 3200
    nb0 = EH // BE
    out0 = pl.pallas_call(
        _edge_final_kernel,
        grid=(nb0,),
        in_specs=[
            pl.BlockSpec((BE, DOUT), lambda i: (i, 0)),
            pl.BlockSpec((BE, DE), lambda i: (i, 0)),
            pl.BlockSpec((DE, DOUT), lambda i: (0, 0)),
            pl.BlockSpec((1, DOUT), lambda i: (0, 0)),
        ],
        out_specs=pl.BlockSpec((BE, DOUT), lambda i: (i, 0)),
        out_shape=jax.ShapeDtypeStruct((E, DOUT), jnp.float32),
    )(G0, edge_attr, W3, b2)

    out = pl.pallas_call(
        _edge_final_kernel2,
        grid=(nb0,),
        in_specs=[
            pl.BlockSpec((BE, DOUT), lambda i: (i, 0)),
            pl.BlockSpec((BE, DE), lambda i: (i + nb0, 0)),
            pl.BlockSpec((DE, DOUT), lambda i: (0, 0)),
            pl.BlockSpec((1, DOUT), lambda i: (0, 0)),
            pl.BlockSpec(memory_space=pl.ANY),
        ],
        out_specs=pl.BlockSpec((BE, DOUT), lambda i: (i + nb0, 0)),
        out_shape=jax.ShapeDtypeStruct((E, DOUT), jnp.float32),
        input_output_aliases={4: 0},
    )(G1, edge_attr, W3, b2, out0)
    return out
